# baseline (speedup 1.0000x reference)
import functools

import jax
import jax.numpy as jnp
from jax import lax
from jax.experimental import pallas as pl
from jax.experimental.pallas import tpu as pltpu
from jax.experimental.pallas import tpu_sc as plsc

N_EMBEDS = 1000000
EMBED_DIM = 64
BATCH = 16384
FIELDS = 26

_NC = 2
_NS = 16
_NW = _NC * _NS
_B = BATCH * FIELDS
_BPW = _B // _NW
_GRP = 128
_CHUNK = 512
_NGRP = _CHUNK // _GRP
_NCHUNK = _BPW // _CHUNK


def _emb_kernel(idx_hbm, table_hbm, out_hbm, idx_v, rows_v, gsem, wsem):
    wid = lax.axis_index("s") * _NC + lax.axis_index("c")
    base = wid * _BPW
    pltpu.sync_copy(idx_hbm.at[pl.ds(base, _BPW)], idx_v)

    def fire_gathers(j, b):
        off = j * _CHUNK
        return [
            pltpu.async_copy(
                table_hbm.at[idx_v.at[pl.ds(off + g * _GRP, _GRP)]],
                rows_v.at[b, pl.ds(g * _GRP, _GRP)],
                gsem.at[b],
            )
            for g in range(_NGRP)
        ]

    gathers = [None, None]
    writes = [None, None]
    for j in range(_NCHUNK):
        b = j % 2
        if writes[b] is not None:
            writes[b].wait()
            writes[b] = None
        gathers[b] = fire_gathers(j, b)
        pb = 1 - b
        if gathers[pb] is not None:
            for c in gathers[pb]:
                c.wait()
            gathers[pb] = None
            writes[pb] = pltpu.async_copy(
                rows_v.at[pb],
                out_hbm.at[pl.ds(base + (j - 1) * _CHUNK, _CHUNK)],
                wsem.at[pb],
            )
    lb = (_NCHUNK - 1) % 2
    for c in gathers[lb]:
        c.wait()
    writes[lb] = pltpu.async_copy(
        rows_v.at[lb],
        out_hbm.at[pl.ds(base + (_NCHUNK - 1) * _CHUNK, _CHUNK)],
        wsem.at[lb],
    )
    for w in writes:
        if w is not None:
            w.wait()


_TROWS = 512
_TGRID = (N_EMBEDS + _TROWS - 1) // _TROWS


def _transpose_body(i_ref, o_ref):
    x = i_ref[...].reshape(EMBED_DIM, _TROWS // 2, 2)
    o_ref[...] = jnp.transpose(x, (1, 2, 0)).reshape(_TROWS // 2, 128)


def _tc_transpose(w_t):
    return pl.pallas_call(
        _transpose_body,
        grid=(_TGRID,),
        in_specs=[pl.BlockSpec((EMBED_DIM, _TROWS), lambda j: (0, j))],
        out_specs=pl.BlockSpec((_TROWS // 2, 128), lambda j: (j, 0)),
        out_shape=jax.ShapeDtypeStruct((N_EMBEDS // 2, 128), jnp.float32),
    )(w_t)


@jax.jit
def _run(idx_flat, weights):
    f = functools.partial(
        pl.kernel,
        mesh=plsc.VectorSubcoreMesh(core_axis_name="c", subcore_axis_name="s"),
        out_type=jax.ShapeDtypeStruct((_B, EMBED_DIM), jnp.float32),
        scratch_types=[
            pltpu.VMEM((_BPW,), jnp.int32),
            pltpu.VMEM((2, _CHUNK, EMBED_DIM), jnp.float32),
            pltpu.SemaphoreType.DMA((2,)),
            pltpu.SemaphoreType.DMA((2,)),
        ],
        compiler_params=pltpu.CompilerParams(use_tc_tiling_on_sc=False),
    )(_emb_kernel)
    return f(idx_flat, weights)


def kernel(x, weights):
    table_lin = _tc_transpose(weights.T).reshape(N_EMBEDS, EMBED_DIM)
    out = _run(x.reshape(-1), table_lin)
    return out.reshape(BATCH, FIELDS, EMBED_DIM)


# speedup vs baseline: 12.0982x; 12.0982x over previous
import functools

import jax
import jax.numpy as jnp
from jax import lax
from jax.experimental import pallas as pl
from jax.experimental.pallas import tpu as pltpu
from jax.experimental.pallas import tpu_sc as plsc

N_EMBEDS = 1000000
EMBED_DIM = 64
BATCH = 16384
FIELDS = 26

_NC = 2
_NS = 16
_NW = _NC * _NS
_B = BATCH * FIELDS
_BPW = _B // _NW
_GRP = 128
_CHUNK = 512
_NGRP = _CHUNK // _GRP
_NCHUNK = _BPW // _CHUNK


def _emb_kernel(idx_hbm, table_hbm, out_hbm, idx_v, rows_v, gsem, wsem):
    wid = lax.axis_index("s") * _NC + lax.axis_index("c")
    base = wid * _BPW
    pltpu.sync_copy(idx_hbm.at[pl.ds(base, _BPW)], idx_v)

    def fire_gathers(j, b):
        off = j * _CHUNK
        return [
            pltpu.async_copy(
                table_hbm.at[idx_v.at[pl.ds(off + g * _GRP, _GRP)]],
                rows_v.at[b, pl.ds(g * _GRP, _GRP)],
                gsem.at[b],
            )
            for g in range(_NGRP)
        ]

    gathers = [None, None]
    writes = [None, None]
    for j in range(_NCHUNK):
        b = j % 2
        if writes[b] is not None:
            writes[b].wait()
            writes[b] = None
        gathers[b] = fire_gathers(j, b)
        pb = 1 - b
        if gathers[pb] is not None:
            for c in gathers[pb]:
                c.wait()
            gathers[pb] = None
            writes[pb] = pltpu.async_copy(
                rows_v.at[pb],
                out_hbm.at[pl.ds(base + (j - 1) * _CHUNK, _CHUNK)],
                wsem.at[pb],
            )
    lb = (_NCHUNK - 1) % 2
    for c in gathers[lb]:
        c.wait()
    writes[lb] = pltpu.async_copy(
        rows_v.at[lb],
        out_hbm.at[pl.ds(base + (_NCHUNK - 1) * _CHUNK, _CHUNK)],
        wsem.at[lb],
    )
    for w in writes:
        if w is not None:
            w.wait()


_TROWS = 8192
_TGRID = (N_EMBEDS + _TROWS - 1) // _TROWS


def _transpose_body(i_ref, o_ref):
    t3 = i_ref[...].T.reshape(_TROWS // 2, 2, EMBED_DIM)
    o_ref[...] = jnp.concatenate([t3[:, 0, :], t3[:, 1, :]], axis=1)


def _tc_transpose(w_t):
    return pl.pallas_call(
        _transpose_body,
        grid=(_TGRID,),
        in_specs=[pl.BlockSpec((EMBED_DIM, _TROWS), lambda j: (0, j))],
        out_specs=pl.BlockSpec((_TROWS // 2, 128), lambda j: (j, 0)),
        out_shape=jax.ShapeDtypeStruct((N_EMBEDS // 2, 128), jnp.float32),
    )(w_t)


@jax.jit
def _run(idx_flat, weights):
    f = functools.partial(
        pl.kernel,
        mesh=plsc.VectorSubcoreMesh(core_axis_name="c", subcore_axis_name="s"),
        out_type=jax.ShapeDtypeStruct((_B, EMBED_DIM), jnp.float32),
        scratch_types=[
            pltpu.VMEM((_BPW,), jnp.int32),
            pltpu.VMEM((2, _CHUNK, EMBED_DIM), jnp.float32),
            pltpu.SemaphoreType.DMA((2,)),
            pltpu.SemaphoreType.DMA((2,)),
        ],
        compiler_params=pltpu.CompilerParams(use_tc_tiling_on_sc=False),
    )(_emb_kernel)
    return f(idx_flat, weights)


def kernel(x, weights):
    table_lin = _tc_transpose(weights.T).reshape(N_EMBEDS, EMBED_DIM)
    out = _run(x.reshape(-1), table_lin)
    return out.reshape(BATCH, FIELDS, EMBED_DIM)
